# baseline (speedup 1.0000x reference)
import functools

import jax
import jax.numpy as jnp
from jax import lax
from jax.experimental import pallas as pl
from jax.experimental.pallas import tpu as pltpu

B = 32
D = 768
N = 32768
K = 256
BLK = 2048
NB = N // BLK

_MASK31 = 0x7FFFFFFF
_INT_MIN = -2147483648
_INT_MAX = 2147483647


def _to_key(v):
    b = lax.bitcast_convert_type(v, jnp.int32)
    return b ^ ((b >> 31) & _MASK31)


def _from_key(k):
    return lax.bitcast_convert_type(k ^ ((k >> 31) & _MASK31), jnp.float32)


def _fused_body(x_ref, we_ref, be_ref, wd_ref, recon_ref, sp_ref,
                key_ref, cmax_ref, thr_ref, midx_ref):
    i = pl.program_id(0)
    nch = BLK // 128

    @pl.when(i < NB)
    def _encode():
        blk = lax.dot_general(x_ref[...], we_ref[...],
                              (((1,), (1,)), ((), ())),
                              preferred_element_type=jnp.float32)
        blk = blk + be_ref[...]
        key_ref[:, pl.ds(i * BLK, BLK)] = _to_key(blk)
        cmax_ref[i] = jnp.max(blk.reshape(B, nch, 128), axis=2)

    @pl.when(i == NB)
    def _threshold():
        keys = key_ref[...]
        ones_col = jnp.ones((N, 1), jnp.bfloat16)

        def _count(mask):
            return lax.dot_general(mask.astype(jnp.bfloat16), ones_col,
                                   (((1,), (0,)), ((), ())),
                                   preferred_element_type=jnp.float32)

        cmaxs = cmax_ref[...]
        lo0 = _to_key(jnp.min(jnp.min(cmaxs, axis=0), axis=1,
                              keepdims=True))
        hi0 = _to_key(jnp.max(jnp.max(cmaxs, axis=0), axis=1,
                              keepdims=True))

        def vcond(carry):
            lo, hi = carry
            return jnp.any(lo < hi)

        def vbody(carry):
            lo, hi = carry
            mid = (lo >> 1) + (hi >> 1) + (lo & hi & 1) + ((lo ^ hi) & 1)
            pred = _count(keys >= mid) >= float(K)
            return (jnp.where(pred, mid, lo),
                    jnp.where(pred, hi, mid - 1))

        thr, _ = lax.while_loop(vcond, vbody, (lo0, hi0))
        thr_ref[...] = thr

        needed = float(K) - _count(keys > thr)
        eq = keys == thr
        cols = lax.broadcasted_iota(jnp.int32, (B, N), 1)
        midx_fast = jnp.min(jnp.where(eq, cols, N - 1), axis=1,
                            keepdims=True)
        resolved = needed <= 1.0
        ilo0 = jnp.where(resolved, midx_fast, 0)
        ihi0 = jnp.where(resolved, midx_fast, N - 1)

        def icond(carry):
            lo, hi = carry
            return jnp.any(lo < hi)

        def ibody(carry):
            lo, hi = carry
            mid = (lo + hi) >> 1
            pred = _count(eq & (cols <= mid)) >= needed
            return (jnp.where(pred, lo, mid + 1),
                    jnp.where(pred, mid, hi))

        midx, _ = lax.while_loop(icond, ibody, (ilo0, ihi0))
        midx_ref[...] = midx

    @pl.when(i >= NB)
    def _decode():
        j = i - NB
        kblk = key_ref[:, pl.ds(j * BLK, BLK)]
        thr = thr_ref[...]
        midx = midx_ref[...]
        cols = lax.broadcasted_iota(jnp.int32, (B, BLK), 1) + j * BLK
        sel = (kblk > thr) | ((kblk == thr) & (cols <= midx))
        sp = jnp.where(sel & (kblk > 0), _from_key(kblk), 0.0)
        sp_ref[...] = sp
        part = lax.dot_general(sp, wd_ref[...],
                               (((1,), (1,)), ((), ())),
                               preferred_element_type=jnp.float32)

        @pl.when(j == 0)
        def _():
            recon_ref[...] = part

        @pl.when(j > 0)
        def _():
            recon_ref[...] = recon_ref[...] + part


@jax.jit
def kernel(x, W_enc, b_enc, W_dec):
    b2 = b_enc.reshape(1, N)
    grid = (2 * NB,)
    recon, sparse = pl.pallas_call(
        _fused_body,
        grid=grid,
        in_specs=[
            pl.BlockSpec((B, D), lambda i: (0, 0)),
            pl.BlockSpec((BLK, D), lambda i: (jnp.minimum(i, NB - 1), 0)),
            pl.BlockSpec((1, BLK), lambda i: (0, jnp.minimum(i, NB - 1))),
            pl.BlockSpec((D, BLK), lambda i: (0, jnp.maximum(i - NB, 0))),
        ],
        out_specs=[
            pl.BlockSpec((B, D), lambda i: (0, 0)),
            pl.BlockSpec((B, BLK), lambda i: (0, jnp.maximum(i - NB, 0))),
        ],
        out_shape=[
            jax.ShapeDtypeStruct((B, D), jnp.float32),
            jax.ShapeDtypeStruct((B, N), jnp.float32),
        ],
        scratch_shapes=[
            pltpu.VMEM((B, N), jnp.int32),
            pltpu.VMEM((NB, B, BLK // 128), jnp.float32),
            pltpu.VMEM((B, 1), jnp.int32),
            pltpu.VMEM((B, 1), jnp.int32),
        ],
    )(x, W_enc, b2, W_dec)
    return recon, sparse


# speedup vs baseline: 1.2060x; 1.2060x over previous
import functools

import jax
import jax.numpy as jnp
from jax import lax
from jax.experimental import pallas as pl
from jax.experimental.pallas import tpu as pltpu

B = 32
D = 768
N = 32768
K = 256
BLK = 2048
NB = N // BLK

_MASK31 = 0x7FFFFFFF
_INT_MIN = -2147483648
_INT_MAX = 2147483647


def _to_key(v):
    b = lax.bitcast_convert_type(v, jnp.int32)
    return b ^ ((b >> 31) & _MASK31)


def _from_key(k):
    return lax.bitcast_convert_type(k ^ ((k >> 31) & _MASK31), jnp.float32)


def _fused_body(x_ref, we_ref, be_ref, wd_ref, recon_ref, sp_ref,
                key_ref, cmax_ref, thr_ref, midx_ref):
    i = pl.program_id(0)
    nch = BLK // 128

    @pl.when(i < NB)
    def _encode():
        blk = lax.dot_general(x_ref[...], we_ref[...],
                              (((1,), (1,)), ((), ())),
                              preferred_element_type=jnp.float32)
        blk = blk + be_ref[...]
        key_ref[:, pl.ds(i * BLK, BLK)] = _to_key(blk)
        cmax_ref[i] = jnp.max(blk.reshape(B, nch, 128), axis=2)

    @pl.when(i == NB)
    def _threshold():
        keys = key_ref[...]

        def _count(mask):
            return jnp.sum(mask.astype(jnp.int32), axis=1, keepdims=True)

        cmaxs = cmax_ref[...]
        lo0 = _to_key(jnp.min(jnp.min(cmaxs, axis=0), axis=1,
                              keepdims=True))
        hi0 = _to_key(jnp.max(jnp.max(cmaxs, axis=0), axis=1,
                              keepdims=True))

        def vcond(carry):
            lo, hi = carry
            return jnp.any(lo < hi)

        def vbody(carry):
            lo, hi = carry
            mid = (lo >> 1) + (hi >> 1) + (lo & hi & 1) + ((lo ^ hi) & 1)
            pred = _count(keys >= mid) >= K
            return (jnp.where(pred, mid, lo),
                    jnp.where(pred, hi, mid - 1))

        thr, _ = lax.while_loop(vcond, vbody, (lo0, hi0))
        thr_ref[...] = thr

        needed = K - _count(keys > thr)
        eq = keys == thr
        cols = lax.broadcasted_iota(jnp.int32, (B, N), 1)
        midx_fast = jnp.min(jnp.where(eq, cols, N - 1), axis=1,
                            keepdims=True)
        resolved = needed <= 1
        ilo0 = jnp.where(resolved, midx_fast, 0)
        ihi0 = jnp.where(resolved, midx_fast, N - 1)

        def icond(carry):
            lo, hi = carry
            return jnp.any(lo < hi)

        def ibody(carry):
            lo, hi = carry
            mid = (lo + hi) >> 1
            pred = _count(eq & (cols <= mid)) >= needed
            return (jnp.where(pred, lo, mid + 1),
                    jnp.where(pred, mid, hi))

        midx, _ = lax.while_loop(icond, ibody, (ilo0, ihi0))
        midx_ref[...] = midx

    @pl.when(i >= NB)
    def _decode():
        j = i - NB
        kblk = key_ref[:, pl.ds(j * BLK, BLK)]
        thr = thr_ref[...]
        midx = midx_ref[...]
        cols = lax.broadcasted_iota(jnp.int32, (B, BLK), 1) + j * BLK
        sel = (kblk > thr) | ((kblk == thr) & (cols <= midx))
        sp = jnp.where(sel & (kblk > 0), _from_key(kblk), 0.0)
        sp_ref[...] = sp
        part = lax.dot_general(sp, wd_ref[...],
                               (((1,), (1,)), ((), ())),
                               preferred_element_type=jnp.float32)

        @pl.when(j == 0)
        def _():
            recon_ref[...] = part

        @pl.when(j > 0)
        def _():
            recon_ref[...] = recon_ref[...] + part


@jax.jit
def kernel(x, W_enc, b_enc, W_dec):
    b2 = b_enc.reshape(1, N)
    grid = (2 * NB,)
    recon, sparse = pl.pallas_call(
        _fused_body,
        grid=grid,
        in_specs=[
            pl.BlockSpec((B, D), lambda i: (0, 0)),
            pl.BlockSpec((BLK, D), lambda i: (jnp.minimum(i, NB - 1), 0)),
            pl.BlockSpec((1, BLK), lambda i: (0, jnp.minimum(i, NB - 1))),
            pl.BlockSpec((D, BLK), lambda i: (0, jnp.maximum(i - NB, 0))),
        ],
        out_specs=[
            pl.BlockSpec((B, D), lambda i: (0, 0)),
            pl.BlockSpec((B, BLK), lambda i: (0, jnp.maximum(i - NB, 0))),
        ],
        out_shape=[
            jax.ShapeDtypeStruct((B, D), jnp.float32),
            jax.ShapeDtypeStruct((B, N), jnp.float32),
        ],
        scratch_shapes=[
            pltpu.VMEM((B, N), jnp.int32),
            pltpu.VMEM((NB, B, BLK // 128), jnp.float32),
            pltpu.VMEM((B, 1), jnp.int32),
            pltpu.VMEM((B, 1), jnp.int32),
        ],
    )(x, W_enc, b2, W_dec)
    return recon, sparse
